# Initial kernel scaffold; baseline (speedup 1.0000x reference)
import jax
import jax.numpy as jnp
from jax.experimental import pallas as pl


def kernel(coords, type_indices, res_ids, pot_tensor):
    raise NotImplementedError("write your pallas kernel here")


# speedup vs baseline: 1217.6266x; 1217.6266x over previous
import functools

import jax
import jax.numpy as jnp
from jax.experimental import pallas as pl
from jax.experimental.pallas import tpu as pltpu

_N = 4096
_NT = 32
_BINS = 28
_B = 512
_NBLK = _N // _B
_INV_BIN = 1.0 / 0.7
_DCUT = 19.6
_SEP_EXCL = 2.0


def _score_kernel(xi, xjt, sqi, sqjt, ri, rjt, tit, tj, pot, out, hist):
    bi = pl.program_id(0)
    bj = pl.program_id(1)

    @pl.when((bi == 0) & (bj == 0))
    def _init():
        hist[...] = jnp.zeros_like(hist)

    g = jax.lax.dot_general(
        xi[...], xjt[...], (((1,), (0,)), ((), ())),
        preferred_element_type=jnp.float32)
    d2 = sqi[...] + sqjt[...] - 2.0 * g
    dist = jnp.sqrt(jnp.maximum(d2, 0.0)) + 1e-8

    sep = jnp.abs(ri[...] - rjt[...])
    mask = (sep > _SEP_EXCL) & (dist < _DCUT)

    ds = dist * _INV_BIN
    d0f = jnp.floor(jnp.minimum(ds, 27.0))
    alpha = ds - d0f
    w0 = jnp.where(mask, 1.0 - alpha, 0.0)
    w1 = jnp.where(mask, alpha, 0.0)
    w01 = jnp.where(mask, 1.0, 0.0)

    zero = jnp.zeros_like(w0)
    m_prev = None
    for b in range(_BINS):
        m_b = d0f == float(b)
        if b == 0:
            a_b = jnp.where(m_b, w0, zero)
        elif b < _BINS - 1:
            a_b = jnp.where(m_b, w0, jnp.where(m_prev, w1, zero))
        else:
            a_b = jnp.where(m_b, w01, jnp.where(m_prev, w1, zero))
        m_prev = m_b
        u = jax.lax.dot_general(
            tit[...], a_b, (((1,), (0,)), ((), ())),
            preferred_element_type=jnp.float32)
        m32 = jax.lax.dot_general(
            u, tj[...], (((1,), (0,)), ((), ())),
            preferred_element_type=jnp.float32)
        hist[b * _NT:(b + 1) * _NT, :] += m32

    @pl.when((bi == _NBLK - 1) & (bj == _NBLK - 1))
    def _fin():
        out[...] = 0.5 * jnp.sum(hist[...] * pot[...], keepdims=True)


@jax.jit
def _run(x, xt, sq, sqt, rf, rft, tit, tj, pot_r):
    out = pl.pallas_call(
        _score_kernel,
        grid=(_NBLK, _NBLK),
        in_specs=[
            pl.BlockSpec((_B, 8), lambda i, j: (i, 0)),
            pl.BlockSpec((8, _B), lambda i, j: (0, j)),
            pl.BlockSpec((_B, 1), lambda i, j: (i, 0)),
            pl.BlockSpec((1, _B), lambda i, j: (0, j)),
            pl.BlockSpec((_B, 1), lambda i, j: (i, 0)),
            pl.BlockSpec((1, _B), lambda i, j: (0, j)),
            pl.BlockSpec((_NT, _B), lambda i, j: (0, i)),
            pl.BlockSpec((_B, _NT), lambda i, j: (j, 0)),
            pl.BlockSpec((_BINS * _NT, _NT), lambda i, j: (0, 0)),
        ],
        out_specs=pl.BlockSpec((1, 1), lambda i, j: (0, 0)),
        out_shape=jax.ShapeDtypeStruct((1, 1), jnp.float32),
        scratch_shapes=[pltpu.VMEM((_BINS * _NT, _NT), jnp.float32)],
    )(x, xt, sq, sqt, rf, rft, tit, tj, pot_r)
    return out[0, 0]


def kernel(coords, type_indices, res_ids, pot_tensor):
    x = jnp.pad(coords.astype(jnp.float32), ((0, 0), (0, 5)))
    xt = x.T
    sq = jnp.sum(x * x, axis=1, keepdims=True)
    sqt = sq.T
    rf = res_ids.astype(jnp.float32).reshape(_N, 1)
    rft = rf.T
    onehot = (type_indices.reshape(_N, 1) ==
              jnp.arange(_NT, dtype=type_indices.dtype).reshape(1, _NT)
              ).astype(jnp.float32)
    tit = onehot.T
    pot_r = jnp.transpose(pot_tensor.astype(jnp.float32), (2, 0, 1)).reshape(
        _BINS * _NT, _NT)
    return _run(x, xt, sq, sqt, rf, rft, tit, onehot, pot_r)


# speedup vs baseline: 2119.4391x; 1.7406x over previous
import functools

import numpy as np
import jax
import jax.numpy as jnp
from jax.experimental import pallas as pl
from jax.experimental.pallas import tpu as pltpu

_N = 4096
_NT = 32
_BINS = 28
_B = 512
_NBLK = _N // _B
_NSTEP = _NBLK * (_NBLK + 1) // 2
_INV_BIN = 1.0 / 0.7
_DCUT = 19.6
_SEP_EXCL = 2.0

_BMAP = np.array(
    [[bi, bj] for bi in range(_NBLK) for bj in range(bi, _NBLK)],
    dtype=np.int32).T


def _score_kernel(bmap, xi, xjt, sqi, sqjt, ri, rjt, tit, tj, pot, out, hist):
    g = pl.program_id(0)
    bi = bmap[0, g]
    bj = bmap[1, g]

    @pl.when(g == 0)
    def _init():
        hist[...] = jnp.zeros_like(hist)

    gm = jax.lax.dot_general(
        xi[...], xjt[...], (((1,), (0,)), ((), ())),
        preferred_element_type=jnp.float32)
    d2 = sqi[...] + sqjt[...] - 2.0 * gm
    dist = jnp.sqrt(jnp.maximum(d2, 0.0)) + 1e-8

    sep = jnp.abs(ri[...] - rjt[...])
    mask = (sep > _SEP_EXCL) & (dist < _DCUT)

    ds = dist * _INV_BIN
    d0f = jnp.floor(jnp.minimum(ds, 27.0))
    alpha = ds - d0f
    scale = jnp.where(bi == bj, 0.5, 1.0)
    wa = jnp.where(mask, scale, 0.0)
    w1 = wa * alpha
    w0 = wa - w1

    zero = jnp.zeros_like(w0)
    m_prev = None
    for b in range(_BINS):
        m_b = d0f == float(b)
        if b == 0:
            a_b = jnp.where(m_b, w0, zero)
        elif b < _BINS - 1:
            a_b = jnp.where(m_b, w0, jnp.where(m_prev, w1, zero))
        else:
            a_b = jnp.where(m_b, wa, jnp.where(m_prev, w1, zero))
        m_prev = m_b
        u = jax.lax.dot_general(
            tit[...], a_b, (((1,), (0,)), ((), ())),
            preferred_element_type=jnp.float32)
        m32 = jax.lax.dot_general(
            u, tj[...], (((1,), (0,)), ((), ())),
            preferred_element_type=jnp.float32)
        hist[b * _NT:(b + 1) * _NT, :] += m32

    @pl.when(g == _NSTEP - 1)
    def _fin():
        out[...] = jnp.sum(hist[...] * pot[...], keepdims=True)


@jax.jit
def _run(x, xt, sq, sqt, rf, rft, tit, tj, pot_r):
    out = pl.pallas_call(
        _score_kernel,
        grid_spec=pltpu.PrefetchScalarGridSpec(
            num_scalar_prefetch=1,
            grid=(_NSTEP,),
            in_specs=[
                pl.BlockSpec((_B, 8), lambda g, m: (m[0, g], 0)),
                pl.BlockSpec((8, _B), lambda g, m: (0, m[1, g])),
                pl.BlockSpec((_B, 1), lambda g, m: (m[0, g], 0)),
                pl.BlockSpec((1, _B), lambda g, m: (0, m[1, g])),
                pl.BlockSpec((_B, 1), lambda g, m: (m[0, g], 0)),
                pl.BlockSpec((1, _B), lambda g, m: (0, m[1, g])),
                pl.BlockSpec((_NT, _B), lambda g, m: (0, m[0, g])),
                pl.BlockSpec((_B, _NT), lambda g, m: (m[1, g], 0)),
                pl.BlockSpec((_BINS * _NT, _NT), lambda g, m: (0, 0)),
            ],
            out_specs=pl.BlockSpec((1, 1), lambda g, m: (0, 0)),
            scratch_shapes=[pltpu.VMEM((_BINS * _NT, _NT), jnp.float32)],
        ),
        out_shape=jax.ShapeDtypeStruct((1, 1), jnp.float32),
    )(jnp.asarray(_BMAP), x, xt, sq, sqt, rf, rft, tit, tj, pot_r)
    return out[0, 0]


def kernel(coords, type_indices, res_ids, pot_tensor):
    x = jnp.pad(coords.astype(jnp.float32), ((0, 0), (0, 5)))
    xt = x.T
    sq = jnp.sum(x * x, axis=1, keepdims=True)
    sqt = sq.T
    rf = res_ids.astype(jnp.float32).reshape(_N, 1)
    rft = rf.T
    onehot = (type_indices.reshape(_N, 1) ==
              jnp.arange(_NT, dtype=type_indices.dtype).reshape(1, _NT)
              ).astype(jnp.float32)
    tit = onehot.T
    pot_r = jnp.transpose(pot_tensor.astype(jnp.float32), (2, 0, 1)).reshape(
        _BINS * _NT, _NT)
    return _run(x, xt, sq, sqt, rf, rft, tit, onehot, pot_r)
